# baseline (speedup 1.0000x reference)
import functools

import jax
import jax.numpy as jnp
from jax import lax
from jax.experimental import pallas as pl
from jax.experimental.pallas import tpu as pltpu
from jax.experimental.pallas import tpu_sc as plsc

_DIM = 32
_NE = 8192
_B = 8192
_BS = 2048
_CH = 2048

_NC = 2
_NS = 16
_NW = _NC * _NS
_BPW = _B // _NW


def _argmin_body(x2t_ref, e_ref, xs_ref, es_ref, idx_ref, av_ref, ai_ref):
    t = pl.program_id(1)
    mm2 = lax.dot_general(e_ref[...], x2t_ref[...], (((0,), (0,)), ((), ())),
                          preferred_element_type=jnp.float32)
    d = (xs_ref[...] - mm2) + es_ref[...]
    m = jnp.min(d, axis=0, keepdims=True)
    iot = lax.broadcasted_iota(jnp.int32, (_CH, _BS), 0)
    cand = jnp.where(d == m, iot, _CH)
    i_t = jnp.min(cand, axis=0, keepdims=True) + t * _CH

    @pl.when(t == 0)
    def _():
        av_ref[...] = jnp.full((1, _BS), jnp.inf, jnp.float32)
        ai_ref[...] = jnp.full((1, _BS), _NE, jnp.int32)

    acc_v = av_ref[...]
    acc_i = ai_ref[...]
    lt = m < acc_v
    take = lt | ((m == acc_v) & (i_t < acc_i))
    vnew = jnp.where(lt, m, acc_v)
    av_ref[...] = vnew.astype(jnp.bfloat16).astype(jnp.float32)
    acc_i = jnp.where(take, i_t, acc_i)
    ai_ref[...] = acc_i

    @pl.when(t == _NE // _CH - 1)
    def _():
        idx_ref[...] = acc_i


_argmin_call = pl.pallas_call(
    _argmin_body,
    grid=(_B // _BS, _NE // _CH),
    in_specs=[
        pl.BlockSpec((_DIM, _BS), lambda i, t: (0, i)),
        pl.BlockSpec((_DIM, _CH), lambda i, t: (0, t)),
        pl.BlockSpec((1, _BS), lambda i, t: (0, i)),
        pl.BlockSpec((_CH, 1), lambda i, t: (t, 0)),
    ],
    out_specs=pl.BlockSpec((1, _BS), lambda i, t: (0, i)),
    out_shape=jax.ShapeDtypeStruct((1, _B), jnp.int32),
    scratch_shapes=[
        pltpu.VMEM((1, _BS), jnp.float32),
        pltpu.VMEM((1, _BS), jnp.int32),
    ],
)


@functools.cache
def _make_gather():
    mesh = plsc.VectorSubcoreMesh(core_axis_name="c", subcore_axis_name="s")

    @functools.partial(
        pl.kernel,
        mesh=mesh,
        compiler_params=pltpu.CompilerParams(use_tc_tiling_on_sc=False),
        out_type=jax.ShapeDtypeStruct((_B, _DIM), jnp.float32),
        scratch_types=[
            pltpu.VMEM((_BPW,), jnp.int32),
            pltpu.VMEM((_BPW, _DIM), jnp.float32),
            pltpu.SemaphoreType.DMA,
        ],
    )
    def _gather(table_hbm, idx_hbm, out_hbm, idx_v, rows_v, sem):
        wid = lax.axis_index("s") * _NC + lax.axis_index("c")
        base = wid * _BPW
        pltpu.sync_copy(idx_hbm.at[pl.ds(base, _BPW)], idx_v)
        pltpu.async_copy(table_hbm.at[idx_v], rows_v, sem).wait()
        pltpu.sync_copy(rows_v, out_hbm.at[pl.ds(base, _BPW)])

    return _gather


def kernel(x, embed):
    n, c, h, w = x.shape
    x2t = (2.0 * x).astype(jnp.bfloat16).reshape(n, c, h * w)
    x2t = jnp.transpose(x2t, (1, 0, 2)).reshape(_DIM, -1)
    xs = jnp.sum(x ** 2, axis=1).reshape(1, -1)
    es = jnp.sum(embed ** 2, axis=0).reshape(-1, 1)
    idx = _argmin_call(x2t, embed, xs, es).reshape(-1)
    table = embed.T
    quant = _make_gather()(table, idx)
    out = jnp.transpose(quant.reshape(n, h, w, c), (0, 3, 1, 2))
    return (out, out)


# speedup vs baseline: 1.1896x; 1.1896x over previous
import functools

import jax
import jax.numpy as jnp
from jax import lax
from jax.experimental import pallas as pl
from jax.experimental.pallas import tpu as pltpu
from jax.experimental.pallas import tpu_sc as plsc

_DIM = 32
_NE = 8192
_B = 8192
_BS = 2048
_CH = 2048

_NC = 2
_NS = 16
_NW = _NC * _NS
_BPW = _B // _NW


def _argmin_body(x2t_ref, e_ref, xs_ref, es_ref, idx_ref, av_ref, ai_ref):
    t = pl.program_id(1)
    mm2 = lax.dot_general(e_ref[...], x2t_ref[...], (((0,), (0,)), ((), ())),
                          preferred_element_type=jnp.float32)
    d = (xs_ref[...] - mm2) + es_ref[...]
    m = jnp.min(d, axis=0, keepdims=True)
    i_t = jnp.argmin(d, axis=0, keepdims=True).astype(jnp.int32) + t * _CH

    @pl.when(t == 0)
    def _():
        av_ref[...] = jnp.full((1, _BS), jnp.inf, jnp.float32)
        ai_ref[...] = jnp.full((1, _BS), _NE, jnp.int32)

    acc_v = av_ref[...]
    acc_i = ai_ref[...]
    lt = m < acc_v
    take = lt | ((m == acc_v) & (i_t < acc_i))
    vnew = jnp.where(lt, m, acc_v)
    av_ref[...] = vnew.astype(jnp.bfloat16).astype(jnp.float32)
    acc_i = jnp.where(take, i_t, acc_i)
    ai_ref[...] = acc_i

    @pl.when(t == _NE // _CH - 1)
    def _():
        idx_ref[...] = acc_i


_argmin_call = pl.pallas_call(
    _argmin_body,
    grid=(_B // _BS, _NE // _CH),
    in_specs=[
        pl.BlockSpec((_DIM, _BS), lambda i, t: (0, i)),
        pl.BlockSpec((_DIM, _CH), lambda i, t: (0, t)),
        pl.BlockSpec((1, _BS), lambda i, t: (0, i)),
        pl.BlockSpec((_CH, 1), lambda i, t: (t, 0)),
    ],
    out_specs=pl.BlockSpec((1, _BS), lambda i, t: (0, i)),
    out_shape=jax.ShapeDtypeStruct((1, _B), jnp.int32),
    scratch_shapes=[
        pltpu.VMEM((1, _BS), jnp.float32),
        pltpu.VMEM((1, _BS), jnp.int32),
    ],
)


@functools.cache
def _make_gather():
    mesh = plsc.VectorSubcoreMesh(core_axis_name="c", subcore_axis_name="s")

    @functools.partial(
        pl.kernel,
        mesh=mesh,
        compiler_params=pltpu.CompilerParams(use_tc_tiling_on_sc=False),
        out_type=jax.ShapeDtypeStruct((_B, _DIM), jnp.float32),
        scratch_types=[
            pltpu.VMEM((_BPW,), jnp.int32),
            pltpu.VMEM((_BPW, _DIM), jnp.float32),
            pltpu.SemaphoreType.DMA,
        ],
    )
    def _gather(table_hbm, idx_hbm, out_hbm, idx_v, rows_v, sem):
        wid = lax.axis_index("s") * _NC + lax.axis_index("c")
        base = wid * _BPW
        pltpu.sync_copy(idx_hbm.at[pl.ds(base, _BPW)], idx_v)
        pltpu.async_copy(table_hbm.at[idx_v], rows_v, sem).wait()
        pltpu.sync_copy(rows_v, out_hbm.at[pl.ds(base, _BPW)])

    return _gather


def kernel(x, embed):
    n, c, h, w = x.shape
    x2t = (2.0 * x).astype(jnp.bfloat16).reshape(n, c, h * w)
    x2t = jnp.transpose(x2t, (1, 0, 2)).reshape(_DIM, -1)
    xs = jnp.sum(x ** 2, axis=1).reshape(1, -1)
    es = jnp.sum(embed ** 2, axis=0).reshape(-1, 1)
    idx = _argmin_call(x2t, embed, xs, es).reshape(-1)
    table = embed.T
    quant = _make_gather()(table, idx)
    out = jnp.transpose(quant.reshape(n, h, w, c), (0, 3, 1, 2))
    return (out, out)


# speedup vs baseline: 1.2362x; 1.0392x over previous
import functools

import jax
import jax.numpy as jnp
from jax import lax
from jax.experimental import pallas as pl
from jax.experimental.pallas import tpu as pltpu
from jax.experimental.pallas import tpu_sc as plsc

_DIM = 32
_NE = 8192
_B = 8192
_BS = 4096
_CH = 2048

_NC = 2
_NS = 16
_NW = _NC * _NS
_BPW = _B // _NW


def _argmin_body(x2t_ref, e_ref, xs_ref, es_ref, idx_ref, av_ref, ai_ref):
    t = pl.program_id(1)
    mm2 = lax.dot_general(e_ref[...], x2t_ref[...], (((0,), (0,)), ((), ())),
                          preferred_element_type=jnp.float32)
    d = (xs_ref[...] - mm2) + es_ref[...]
    m = jnp.min(d, axis=0, keepdims=True)
    i_t = jnp.argmin(d, axis=0, keepdims=True).astype(jnp.int32) + t * _CH

    @pl.when(t == 0)
    def _():
        av_ref[...] = jnp.full((1, _BS), jnp.inf, jnp.float32)
        ai_ref[...] = jnp.full((1, _BS), _NE, jnp.int32)

    acc_v = av_ref[...]
    acc_i = ai_ref[...]
    lt = m < acc_v
    take = lt | ((m == acc_v) & (i_t < acc_i))
    vnew = jnp.where(lt, m, acc_v)
    av_ref[...] = vnew.astype(jnp.bfloat16).astype(jnp.float32)
    acc_i = jnp.where(take, i_t, acc_i)
    ai_ref[...] = acc_i

    @pl.when(t == _NE // _CH - 1)
    def _():
        idx_ref[...] = acc_i


_argmin_call = pl.pallas_call(
    _argmin_body,
    grid=(_B // _BS, _NE // _CH),
    in_specs=[
        pl.BlockSpec((_DIM, _BS), lambda i, t: (0, i)),
        pl.BlockSpec((_DIM, _CH), lambda i, t: (0, t)),
        pl.BlockSpec((1, _BS), lambda i, t: (0, i)),
        pl.BlockSpec((_CH, 1), lambda i, t: (t, 0)),
    ],
    out_specs=pl.BlockSpec((1, _BS), lambda i, t: (0, i)),
    out_shape=jax.ShapeDtypeStruct((1, _B), jnp.int32),
    scratch_shapes=[
        pltpu.VMEM((1, _BS), jnp.float32),
        pltpu.VMEM((1, _BS), jnp.int32),
    ],
)


@functools.cache
def _make_gather():
    mesh = plsc.VectorSubcoreMesh(core_axis_name="c", subcore_axis_name="s")

    @functools.partial(
        pl.kernel,
        mesh=mesh,
        compiler_params=pltpu.CompilerParams(use_tc_tiling_on_sc=False),
        out_type=jax.ShapeDtypeStruct((_B, _DIM), jnp.float32),
        scratch_types=[
            pltpu.VMEM((_BPW,), jnp.int32),
            pltpu.VMEM((_BPW, _DIM), jnp.float32),
            pltpu.SemaphoreType.DMA,
        ],
    )
    def _gather(table_hbm, idx_hbm, out_hbm, idx_v, rows_v, sem):
        wid = lax.axis_index("s") * _NC + lax.axis_index("c")
        base = wid * _BPW
        pltpu.sync_copy(idx_hbm.at[pl.ds(base, _BPW)], idx_v)
        pltpu.async_copy(table_hbm.at[idx_v], rows_v, sem).wait()
        pltpu.sync_copy(rows_v, out_hbm.at[pl.ds(base, _BPW)])

    return _gather


def kernel(x, embed):
    n, c, h, w = x.shape
    x2t = (2.0 * x).astype(jnp.bfloat16).reshape(n, c, h * w)
    x2t = jnp.transpose(x2t, (1, 0, 2)).reshape(_DIM, -1)
    xs = jnp.sum(x ** 2, axis=1).reshape(1, -1)
    es = jnp.sum(embed ** 2, axis=0).reshape(-1, 1)
    idx = _argmin_call(x2t, embed, xs, es).reshape(-1)
    table = embed.T
    quant = _make_gather()(table, idx)
    out = jnp.transpose(quant.reshape(n, h, w, c), (0, 3, 1, 2))
    return (out, out)


# speedup vs baseline: 1.2396x; 1.0028x over previous
import functools

import jax
import jax.numpy as jnp
from jax import lax
from jax.experimental import pallas as pl
from jax.experimental.pallas import tpu as pltpu
from jax.experimental.pallas import tpu_sc as plsc

_DIM = 32
_NE = 8192
_B = 8192
_BS = 4096
_HW = 1024
_CH = 2048

_NC = 2
_NS = 16
_NW = _NC * _NS
_BPW = _B // _NW


def _argmin_body(x2t_ref, e_ref, xs_ref, es_ref, idx_ref, av_ref, ai_ref):
    t = pl.program_id(1)
    ms = []
    its = []
    es = es_ref[...]
    for k in range(_BS // _HW):
        mm2 = lax.dot_general(e_ref[...], x2t_ref[k],
                              (((0,), (0,)), ((), ())),
                              preferred_element_type=jnp.float32)
        d = (xs_ref[:, k * _HW:(k + 1) * _HW] - mm2) + es
        ms.append(jnp.min(d, axis=0, keepdims=True))
        its.append(jnp.argmin(d, axis=0, keepdims=True).astype(jnp.int32))
    m = jnp.concatenate(ms, axis=1)
    i_t = jnp.concatenate(its, axis=1) + t * _CH

    @pl.when(t == 0)
    def _():
        av_ref[...] = jnp.full((1, _BS), jnp.inf, jnp.float32)
        ai_ref[...] = jnp.full((1, _BS), _NE, jnp.int32)

    acc_v = av_ref[...]
    acc_i = ai_ref[...]
    lt = m < acc_v
    take = lt | ((m == acc_v) & (i_t < acc_i))
    vnew = jnp.where(lt, m, acc_v)
    av_ref[...] = vnew.astype(jnp.bfloat16).astype(jnp.float32)
    acc_i = jnp.where(take, i_t, acc_i)
    ai_ref[...] = acc_i

    @pl.when(t == _NE // _CH - 1)
    def _():
        idx_ref[...] = acc_i


_argmin_call = pl.pallas_call(
    _argmin_body,
    grid=(_B // _BS, _NE // _CH),
    in_specs=[
        pl.BlockSpec((_BS // _HW, _DIM, _HW), lambda i, t: (i, 0, 0)),
        pl.BlockSpec((_DIM, _CH), lambda i, t: (0, t)),
        pl.BlockSpec((1, _BS), lambda i, t: (0, i)),
        pl.BlockSpec((_CH, 1), lambda i, t: (t, 0)),
    ],
    out_specs=pl.BlockSpec((1, _BS), lambda i, t: (0, i)),
    out_shape=jax.ShapeDtypeStruct((1, _B), jnp.int32),
    scratch_shapes=[
        pltpu.VMEM((1, _BS), jnp.float32),
        pltpu.VMEM((1, _BS), jnp.int32),
    ],
)


@functools.cache
def _make_gather():
    mesh = plsc.VectorSubcoreMesh(core_axis_name="c", subcore_axis_name="s")

    @functools.partial(
        pl.kernel,
        mesh=mesh,
        compiler_params=pltpu.CompilerParams(use_tc_tiling_on_sc=False),
        out_type=jax.ShapeDtypeStruct((_B, _DIM), jnp.float32),
        scratch_types=[
            pltpu.VMEM((_BPW,), jnp.int32),
            pltpu.VMEM((_BPW, _DIM), jnp.float32),
            pltpu.SemaphoreType.DMA,
        ],
    )
    def _gather(table_hbm, idx_hbm, out_hbm, idx_v, rows_v, sem):
        wid = lax.axis_index("s") * _NC + lax.axis_index("c")
        base = wid * _BPW
        pltpu.sync_copy(idx_hbm.at[pl.ds(base, _BPW)], idx_v)
        pltpu.async_copy(table_hbm.at[idx_v], rows_v, sem).wait()
        pltpu.sync_copy(rows_v, out_hbm.at[pl.ds(base, _BPW)])

    return _gather


def kernel(x, embed):
    n, c, h, w = x.shape
    x2t = (2.0 * x).astype(jnp.bfloat16).reshape(n, c, h * w)
    xs = jnp.sum(x ** 2, axis=1).reshape(1, -1)
    es = jnp.sum(embed ** 2, axis=0).reshape(-1, 1)
    idx = _argmin_call(x2t, embed, xs, es).reshape(-1)
    table = embed.T
    quant = _make_gather()(table, idx)
    out = jnp.transpose(quant.reshape(n, h, w, c), (0, 3, 1, 2))
    return (out, out)
